# baseline (speedup 1.0000x reference)
import jax
import jax.numpy as jnp
from jax import lax
from jax.experimental import pallas as pl
from jax.experimental.pallas import tpu as pltpu
from jax.experimental.pallas import tpu_sc as plsc

NC = 2
NS = 16
NW = NC * NS
LANES = 16

VOCAB = 1000000
EMBED = 64
ROWS = 4096
COLS = 200
CHUNK = 128
NBUF = 4
NGROUPS = COLS // NBUF
SCALE = float(EMBED) ** 0.5


def _body(x_hbm, table_hbm, out_hbm, idx_v, buf_v, bufT_v, gsem, osem):
    wid = lax.axis_index("s") * NC + lax.axis_index("c")
    pltpu.sync_copy(x_hbm.at[wid], idx_v)

    iota = lax.broadcasted_iota(jnp.int32, (LANES,), 0)
    row_blocks = [iota + j * LANES for j in range(CHUNK // LANES)]

    def fire_gather(r, rr):
        pltpu.make_async_copy(
            table_hbm.at[idx_v.at[r]], buf_v.at[rr], gsem
        ).start()

    def wait_gather(rr):
        pltpu.make_async_copy(
            table_hbm.at[idx_v.at[0]], buf_v.at[rr], gsem
        ).wait()

    def wait_one_out():
        pltpu.make_async_copy(bufT_v.at[0], out_hbm.at[0, :, wid], osem).wait()

    def transpose_scale(rr):
        def ct_body(ct, carry):
            base = jnp.full((LANES,), ct * 8, jnp.int32)
            for cs in range(8):
                col = base + cs
                for j in range(CHUNK // LANES):
                    v = plsc.load_gather(buf_v.at[rr], [row_blocks[j], col])
                    bufT_v[rr, ct, cs, pl.ds(j * LANES, LANES)] = v * SCALE
            return carry
        lax.fori_loop(0, 8, ct_body, 0)

    def group(g, carry):
        for rr in range(NBUF):
            r = g * NBUF + rr
            wait_gather(rr)
            pl.when(g > 0)(wait_one_out)
            transpose_scale(rr)
            pltpu.make_async_copy(
                bufT_v.at[rr], out_hbm.at[r, :, wid], osem
            ).start()
            pl.when(g < NGROUPS - 1)(lambda: fire_gather(r + NBUF, rr))
        return carry

    for rr in range(NBUF):
        fire_gather(rr, rr)
    lax.fori_loop(0, NGROUPS, group, 0)
    for _ in range(NBUF):
        wait_one_out()


def kernel(x, table):
    xst = x.reshape(NW, CHUNK, COLS).transpose(0, 2, 1).astype(jnp.int32)
    o5 = pl.kernel(
        _body,
        out_type=jax.ShapeDtypeStruct((COLS, 8, NW, 8, CHUNK), jnp.float32),
        mesh=plsc.VectorSubcoreMesh(core_axis_name="c", subcore_axis_name="s"),
        scratch_types=[
            pltpu.VMEM((COLS, CHUNK), jnp.int32),
            pltpu.VMEM((NBUF, CHUNK, EMBED), jnp.float32),
            pltpu.VMEM((NBUF, 8, 8, CHUNK), jnp.float32),
            pltpu.SemaphoreType.DMA,
            pltpu.SemaphoreType.DMA,
        ],
        compiler_params=pltpu.CompilerParams(
            use_tc_tiling_on_sc=False, needs_layout_passes=False
        ),
    )(xst, table)
    return o5.transpose(2, 4, 0, 1, 3).reshape(ROWS, COLS, EMBED)


# speedup vs baseline: 1.7718x; 1.7718x over previous
import jax
import jax.numpy as jnp
from jax import lax
from jax.experimental import pallas as pl
from jax.experimental.pallas import tpu as pltpu
from jax.experimental.pallas import tpu_sc as plsc

NC = 2
NS = 16
NW = NC * NS
LANES = 16

VOCAB = 1000000
EMBED = 64
ROWS = 4096
COLS = 200
CHUNK = 128
NBUF = 4
NGROUPS = COLS // NBUF
SCALE = float(EMBED) ** 0.5


def _body(x_hbm, table_hbm, out_hbm, idx_v, buf_v, bufT_v, gsem, osem):
    wid = lax.axis_index("s") * NC + lax.axis_index("c")
    pltpu.sync_copy(x_hbm.at[wid], idx_v)

    iota = lax.broadcasted_iota(jnp.int32, (LANES,), 0)
    cblocks = [((c0 + iota) // 8, (c0 + iota) % 8) for c0 in range(0, EMBED, LANES)]

    def fire_gather(r, rr):
        pltpu.make_async_copy(
            table_hbm.at[idx_v.at[r]],
            buf_v.at[rr],
            gsem,
        ).start()

    def wait_gather(rr):
        pltpu.make_async_copy(
            table_hbm.at[idx_v.at[0]],
            buf_v.at[rr],
            gsem,
        ).wait()

    def wait_one_out():
        pltpu.make_async_copy(
            bufT_v.at[0, :, :, pl.ds(0, CHUNK)], out_hbm.at[0, :, wid], osem
        ).wait()

    def transpose_scale(rr):
        def a_body(a, carry):
            a_splat = jnp.full((LANES,), a, jnp.int32)
            for cb, (ct_i, cs_i) in enumerate(cblocks):
                v = buf_v[rr, a, pl.ds(cb * LANES, LANES)]
                plsc.store_scatter(
                    bufT_v.at[rr], [ct_i, cs_i, a_splat], v * SCALE
                )
            return carry
        lax.fori_loop(0, CHUNK, a_body, 0, unroll=4)

    def group(g, carry):
        for rr in range(NBUF):
            r = g * NBUF + rr
            wait_gather(rr)
            pl.when(g > 0)(wait_one_out)
            transpose_scale(rr)
            pltpu.make_async_copy(
                bufT_v.at[rr, :, :, pl.ds(0, CHUNK)],
                out_hbm.at[r, :, wid],
                osem,
            ).start()
            pl.when(g < NGROUPS - 1)(lambda: fire_gather(r + NBUF, rr))
        return carry

    for rr in range(NBUF):
        fire_gather(rr, rr)
    lax.fori_loop(0, NGROUPS, group, 0)
    for _ in range(NBUF):
        wait_one_out()


def kernel(x, table):
    xst = x.reshape(NW, CHUNK, COLS).transpose(0, 2, 1).astype(jnp.int32)
    o5 = pl.kernel(
        _body,
        out_type=jax.ShapeDtypeStruct((COLS, 8, NW, 8, CHUNK), jnp.float32),
        mesh=plsc.VectorSubcoreMesh(core_axis_name="c", subcore_axis_name="s"),
        scratch_types=[
            pltpu.VMEM((COLS, CHUNK), jnp.int32),
            pltpu.VMEM((NBUF, CHUNK, EMBED), jnp.float32),
            pltpu.VMEM((NBUF, 8, 8, CHUNK + 1), jnp.float32),
            pltpu.SemaphoreType.DMA,
            pltpu.SemaphoreType.DMA,
        ],
        compiler_params=pltpu.CompilerParams(
            use_tc_tiling_on_sc=False, needs_layout_passes=False
        ),
    )(xst, table)
    return o5.transpose(2, 4, 0, 1, 3).reshape(ROWS, COLS, EMBED)
